# baseline (speedup 1.0000x reference)
import functools

import jax
import jax.numpy as jnp
from jax import lax
from jax.experimental import pallas as pl
from jax.experimental.pallas import tpu as pltpu
from jax.experimental.pallas import tpu_sc as plsc

_NUM_CORES = 2
_NUM_SUBCORES = 16
_NUM_WORKERS = _NUM_CORES * _NUM_SUBCORES
_SPLIT = 6144
_BLOCK_ROWS = 2048
_SC_CHUNK = 32
_SC_NBUF = 2


def _copy_block(in_ref, out_ref):
    out_ref[...] = in_ref[...]


def _tc_copy(table):
    n, d = table.shape
    return pl.pallas_call(
        _copy_block,
        grid=(_SPLIT // _BLOCK_ROWS,),
        in_specs=[pl.BlockSpec((_BLOCK_ROWS, d), lambda i: (i, 0))],
        out_specs=pl.BlockSpec((_BLOCK_ROWS, d), lambda i: (i, 0)),
        out_shape=jax.ShapeDtypeStruct((_SPLIT, d), table.dtype),
    )(table)


def _sc_copy(table):
    n, d = table.shape
    rows = n - _SPLIT
    rows_per_w = rows // _NUM_WORKERS
    nchunk = rows_per_w // _SC_CHUNK

    mesh = plsc.VectorSubcoreMesh(core_axis_name="c", subcore_axis_name="s")

    @functools.partial(
        pl.kernel,
        mesh=mesh,
        out_type=jax.ShapeDtypeStruct((rows, d), table.dtype),
        scratch_types=[
            pltpu.VMEM((_SC_NBUF, _SC_CHUNK, d), table.dtype),
            pltpu.SemaphoreType.DMA((_SC_NBUF,)),
            pltpu.SemaphoreType.DMA((_SC_NBUF,)),
        ],
    )
    def sc_copy(table_hbm, out_hbm, buf, in_sems, out_sems):
        wid = lax.axis_index("s") * _NUM_CORES + lax.axis_index("c")
        base = wid * rows_per_w

        def in_copy(c, b):
            sl = pl.ds(_SPLIT + base + c * _SC_CHUNK, _SC_CHUNK)
            return pltpu.make_async_copy(table_hbm.at[sl], buf.at[b], in_sems.at[b])

        def out_copy(c, b):
            sl = pl.ds(base + c * _SC_CHUNK, _SC_CHUNK)
            return pltpu.make_async_copy(buf.at[b], out_hbm.at[sl], out_sems.at[b])

        for c in range(min(_SC_NBUF, nchunk)):
            in_copy(c, c).start()
        for c in range(nchunk):
            b = c % _SC_NBUF
            in_copy(c, b).wait()
            out_copy(c, b).start()
            prev = c - 1
            nxt = prev + _SC_NBUF
            if prev >= 0 and nxt < nchunk:
                out_copy(prev, prev % _SC_NBUF).wait()
                in_copy(nxt, nxt % _SC_NBUF).start()
        for c in range(max(0, nchunk - _SC_NBUF), nchunk):
            out_copy(c, c % _SC_NBUF).wait()

    return sc_copy(table)


def kernel(table):
    top = _tc_copy(table)
    bottom = _sc_copy(table)
    return jnp.concatenate([top, bottom], axis=0)


# speedup vs baseline: 2.7822x; 2.7822x over previous
import jax
import jax.numpy as jnp
from jax.experimental import pallas as pl
from jax.experimental.pallas import tpu as pltpu

_CHUNK_ROWS = 2048
_NBUF = 4


def _dma_ring(table_hbm, out_hbm, buf, in_sems, out_sems):
    n = table_hbm.shape[0]
    nchunk = n // _CHUNK_ROWS

    def in_copy(c, b):
        sl = pl.ds(c * _CHUNK_ROWS, _CHUNK_ROWS)
        return pltpu.make_async_copy(table_hbm.at[sl], buf.at[b], in_sems.at[b])

    def out_copy(c, b):
        sl = pl.ds(c * _CHUNK_ROWS, _CHUNK_ROWS)
        return pltpu.make_async_copy(buf.at[b], out_hbm.at[sl], out_sems.at[b])

    for c in range(min(_NBUF, nchunk)):
        in_copy(c, c).start()
    for c in range(nchunk):
        b = c % _NBUF
        in_copy(c, b).wait()
        out_copy(c, b).start()
        prev = c - 1
        nxt = prev + _NBUF
        if prev >= 0 and nxt < nchunk:
            out_copy(prev, prev % _NBUF).wait()
            in_copy(nxt, nxt % _NBUF).start()
    for c in range(max(0, nchunk - _NBUF), nchunk):
        out_copy(c, c % _NBUF).wait()


def kernel(table):
    n, d = table.shape
    return pl.pallas_call(
        _dma_ring,
        in_specs=[pl.BlockSpec(memory_space=pl.MemorySpace.ANY)],
        out_specs=pl.BlockSpec(memory_space=pl.MemorySpace.ANY),
        out_shape=jax.ShapeDtypeStruct((n, d), table.dtype),
        scratch_shapes=[
            pltpu.VMEM((_NBUF, _CHUNK_ROWS, d), table.dtype),
            pltpu.SemaphoreType.DMA((_NBUF,)),
            pltpu.SemaphoreType.DMA((_NBUF,)),
        ],
    )(table)
